# baseline (speedup 1.0000x reference)
import jax
import jax.numpy as jnp
from jax.experimental import pallas as pl
from jax.experimental.pallas import tpu as pltpu

BATCH = 16

def _body(emb_any, out_any, scratch, load_sem, sems):
    cp = pltpu.make_async_copy(emb_any, scratch, load_sem)
    cp.start()
    cp.wait()
    for b in range(BATCH):
        if b % 2 == 0:
            pltpu.make_async_copy(scratch, out_any.at[b], sems.at[b]).start()
        else:
            pltpu.make_async_copy(emb_any, out_any.at[b], sems.at[b]).start()
    for b in range(BATCH):
        if b % 2 == 0:
            pltpu.make_async_copy(scratch, out_any.at[b], sems.at[b]).wait()
        else:
            pltpu.make_async_copy(emb_any, out_any.at[b], sems.at[b]).wait()

def kernel(x, grid_embedding):
    batch = x.shape[0]
    g2, f = grid_embedding.shape
    return pl.pallas_call(
        _body,
        in_specs=[pl.BlockSpec(memory_space=pl.ANY)],
        out_specs=pl.BlockSpec(memory_space=pl.ANY),
        out_shape=jax.ShapeDtypeStruct((batch, g2, f), grid_embedding.dtype),
        scratch_shapes=[
            pltpu.VMEM((g2, f), grid_embedding.dtype),
            pltpu.SemaphoreType.DMA,
            pltpu.SemaphoreType.DMA((BATCH,)),
        ],
    )(grid_embedding)


# speedup vs baseline: 10.4425x; 10.4425x over previous
import jax
import jax.numpy as jnp
from jax.experimental import pallas as pl
from jax.experimental.pallas import tpu as pltpu

BATCH = 16

def _body(emb_any, out_any, scratch, load_sem, sems):
    cp = pltpu.make_async_copy(emb_any, scratch, load_sem)
    cp.start()
    cp.wait()
    for b in range(BATCH):
        pltpu.make_async_copy(scratch, out_any.at[b], sems.at[b]).start()
    for b in range(BATCH):
        pltpu.make_async_copy(scratch, out_any.at[b], sems.at[b]).wait()

def kernel(x, grid_embedding):
    batch = x.shape[0]
    g2, f = grid_embedding.shape
    r, c = (g2 * f) // 512, 512
    emb2 = grid_embedding.reshape(r, c)
    out = pl.pallas_call(
        _body,
        in_specs=[pl.BlockSpec(memory_space=pl.ANY)],
        out_specs=pl.BlockSpec(memory_space=pl.ANY),
        out_shape=jax.ShapeDtypeStruct((batch, r, c), grid_embedding.dtype),
        scratch_shapes=[
            pltpu.VMEM((r, c), grid_embedding.dtype),
            pltpu.SemaphoreType.DMA,
            pltpu.SemaphoreType.DMA((BATCH,)),
        ],
    )(emb2)
    return out.reshape(batch, g2, f)


# speedup vs baseline: 86.3044x; 8.2647x over previous
import jax
import jax.numpy as jnp
from jax.experimental import pallas as pl
from jax.experimental.pallas import tpu as pltpu

BATCH = 16

def _body(emb_any, out_any, scratch, load_sem, sems):
    cp = pltpu.make_async_copy(emb_any, scratch, load_sem)
    cp.start()
    cp.wait()
    for b in range(BATCH):
        pltpu.make_async_copy(scratch, out_any.at[b], sems.at[b]).start()
    for b in range(BATCH):
        pltpu.make_async_copy(scratch, out_any.at[b], sems.at[b]).wait()

def kernel(x, grid_embedding):
    batch = x.shape[0]
    g2, f = grid_embedding.shape
    emb_t = grid_embedding.T
    out_t = pl.pallas_call(
        _body,
        in_specs=[pl.BlockSpec(memory_space=pl.ANY)],
        out_specs=pl.BlockSpec(memory_space=pl.ANY),
        out_shape=jax.ShapeDtypeStruct((batch, f, g2), grid_embedding.dtype),
        scratch_shapes=[
            pltpu.VMEM((f, g2), grid_embedding.dtype),
            pltpu.SemaphoreType.DMA,
            pltpu.SemaphoreType.DMA((BATCH,)),
        ],
    )(emb_t)
    return jnp.transpose(out_t, (0, 2, 1))
